# baseline (speedup 1.0000x reference)
import jax
import jax.numpy as jnp
from jax import lax
from jax.experimental import pallas as pl
from jax.experimental.pallas import tpu as pltpu

BLOCK_N = 128


def _fused_body(idx_sref, mem_ref, x_ref, w_in_t_ref, w_h_t_ref, bias_ref,
                gamma_ref, beta_ref, out_ref, curh_scr, nh_scr):
    i = pl.program_id(0)
    b = mem_ref.shape[0]
    out_ref[...] = mem_ref[...]

    def gather_row(r, _):
        idx = idx_sref[i * b + r]
        curh_scr[r, :] = mem_ref[r, idx, :]
        return 0

    lax.fori_loop(0, b, gather_row, 0, unroll=8)

    pre = (jnp.dot(x_ref[...], w_in_t_ref[...], preferred_element_type=jnp.float32)
           + jnp.dot(curh_scr[...], w_h_t_ref[...], preferred_element_type=jnp.float32)
           + bias_ref[...])
    mean = jnp.mean(pre, axis=-1, keepdims=True)
    cent = pre - mean
    var = jnp.mean(cent * cent, axis=-1, keepdims=True)
    nh_scr[...] = jnp.tanh(cent * lax.rsqrt(var + 1e-5) * gamma_ref[...]
                           + beta_ref[...])

    def scatter_row(r, _):
        idx = idx_sref[i * b + r]
        out_ref[r, idx, :] = nh_scr[r, :]
        return 0

    lax.fori_loop(0, b, scatter_row, 0, unroll=8)


def kernel(memory, veh_idx, veh_repr, cust_repr, edge_emb,
           W_in, b_in, W_h, b_h, ln_gamma, ln_beta):
    n, l, h = memory.shape
    d = veh_repr.shape[-1]
    x = jnp.concatenate(
        [veh_repr[:, 0, :], cust_repr[:, 0, :], edge_emb[:, 0, 0, :]], axis=-1)
    w_in_t = W_in.T
    w_h_t = W_h.T
    bias = (b_in + b_h).reshape(1, h)
    gamma = ln_gamma.reshape(1, h)
    beta = ln_beta.reshape(1, h)
    idx = veh_idx.reshape(n).astype(jnp.int32)

    grid_spec = pltpu.PrefetchScalarGridSpec(
        num_scalar_prefetch=1,
        grid=(n // BLOCK_N,),
        in_specs=[
            pl.BlockSpec((BLOCK_N, l, h), lambda i, *_: (i, 0, 0)),
            pl.BlockSpec((BLOCK_N, 3 * d), lambda i, *_: (i, 0)),
            pl.BlockSpec((3 * d, h), lambda i, *_: (0, 0)),
            pl.BlockSpec((h, h), lambda i, *_: (0, 0)),
            pl.BlockSpec((1, h), lambda i, *_: (0, 0)),
            pl.BlockSpec((1, h), lambda i, *_: (0, 0)),
            pl.BlockSpec((1, h), lambda i, *_: (0, 0)),
        ],
        out_specs=pl.BlockSpec((BLOCK_N, l, h), lambda i, *_: (i, 0, 0)),
        scratch_shapes=[
            pltpu.VMEM((BLOCK_N, h), jnp.float32),
            pltpu.VMEM((BLOCK_N, h), jnp.float32),
        ],
    )
    return pl.pallas_call(
        _fused_body,
        grid_spec=grid_spec,
        out_shape=jax.ShapeDtypeStruct((n, l, h), jnp.float32),
    )(idx, memory, x, w_in_t, w_h_t, bias, gamma, beta)


# speedup vs baseline: 1.0194x; 1.0194x over previous
import jax
import jax.numpy as jnp
from jax import lax
from jax.experimental import pallas as pl
from jax.experimental.pallas import tpu as pltpu

BLOCK_N = 512


def _fused_body(idx_sref, mem_ref, x_ref, w_in_t_ref, w_h_t_ref, bias_ref,
                gamma_ref, beta_ref, out_ref, curh_scr, nh_scr):
    i = pl.program_id(0)
    b = mem_ref.shape[0]
    out_ref[...] = mem_ref[...]

    def gather_row(r, _):
        idx = idx_sref[i * b + r]
        curh_scr[r, :] = mem_ref[r, idx, :]
        return 0

    lax.fori_loop(0, b, gather_row, 0, unroll=8)

    pre = (jnp.dot(x_ref[...], w_in_t_ref[...], preferred_element_type=jnp.float32)
           + jnp.dot(curh_scr[...], w_h_t_ref[...], preferred_element_type=jnp.float32)
           + bias_ref[...])
    mean = jnp.mean(pre, axis=-1, keepdims=True)
    cent = pre - mean
    var = jnp.mean(cent * cent, axis=-1, keepdims=True)
    nh_scr[...] = jnp.tanh(cent * lax.rsqrt(var + 1e-5) * gamma_ref[...]
                           + beta_ref[...])

    def scatter_row(r, _):
        idx = idx_sref[i * b + r]
        out_ref[r, idx, :] = nh_scr[r, :]
        return 0

    lax.fori_loop(0, b, scatter_row, 0, unroll=8)


def kernel(memory, veh_idx, veh_repr, cust_repr, edge_emb,
           W_in, b_in, W_h, b_h, ln_gamma, ln_beta):
    n, l, h = memory.shape
    d = veh_repr.shape[-1]
    x = jnp.concatenate(
        [veh_repr[:, 0, :], cust_repr[:, 0, :], edge_emb[:, 0, 0, :]], axis=-1)
    w_in_t = W_in.T
    w_h_t = W_h.T
    bias = (b_in + b_h).reshape(1, h)
    gamma = ln_gamma.reshape(1, h)
    beta = ln_beta.reshape(1, h)
    idx = veh_idx.reshape(n).astype(jnp.int32)

    grid_spec = pltpu.PrefetchScalarGridSpec(
        num_scalar_prefetch=1,
        grid=(n // BLOCK_N,),
        in_specs=[
            pl.BlockSpec((BLOCK_N, l, h), lambda i, *_: (i, 0, 0)),
            pl.BlockSpec((BLOCK_N, 3 * d), lambda i, *_: (i, 0)),
            pl.BlockSpec((3 * d, h), lambda i, *_: (0, 0)),
            pl.BlockSpec((h, h), lambda i, *_: (0, 0)),
            pl.BlockSpec((1, h), lambda i, *_: (0, 0)),
            pl.BlockSpec((1, h), lambda i, *_: (0, 0)),
            pl.BlockSpec((1, h), lambda i, *_: (0, 0)),
        ],
        out_specs=pl.BlockSpec((BLOCK_N, l, h), lambda i, *_: (i, 0, 0)),
        scratch_shapes=[
            pltpu.VMEM((BLOCK_N, h), jnp.float32),
            pltpu.VMEM((BLOCK_N, h), jnp.float32),
        ],
    )
    return pl.pallas_call(
        _fused_body,
        grid_spec=grid_spec,
        out_shape=jax.ShapeDtypeStruct((n, l, h), jnp.float32),
    )(idx, memory, x, w_in_t, w_h_t, bias, gamma, beta)


# speedup vs baseline: 1.2040x; 1.1811x over previous
import jax
import jax.numpy as jnp
from jax import lax
from jax.experimental import pallas as pl
from jax.experimental.pallas import tpu as pltpu


def _gather_mlp_body(idx_sref, mem_any, x_ref, w_in_t_ref, w_h_t_ref,
                     bias_ref, gamma_ref, beta_ref, nh_ref, curh_scr, sem):
    n = nh_ref.shape[0]

    def gather_row(r, _):
        idx = idx_sref[r]
        pltpu.make_async_copy(mem_any.at[r, idx], curh_scr.at[r], sem).start()
        return 0

    lax.fori_loop(0, n, gather_row, 0, unroll=8)
    pltpu.make_async_copy(mem_any.at[0], curh_scr, sem).wait()

    pre = (jnp.dot(x_ref[...], w_in_t_ref[...], preferred_element_type=jnp.float32)
           + jnp.dot(curh_scr[...], w_h_t_ref[...], preferred_element_type=jnp.float32)
           + bias_ref[...])
    mean = jnp.mean(pre, axis=-1, keepdims=True)
    cent = pre - mean
    var = jnp.mean(cent * cent, axis=-1, keepdims=True)
    nh_ref[...] = jnp.tanh(cent * lax.rsqrt(var + 1e-5) * gamma_ref[...]
                           + beta_ref[...])


def _scatter_body(mem_any, nh_ref, idx_sref, out_any, drain_scr, sem):
    n = nh_ref.shape[0]

    def scatter_row(r, _):
        idx = idx_sref[r]
        pltpu.make_async_copy(nh_ref.at[r], out_any.at[r, idx], sem).start()
        return 0

    lax.fori_loop(0, n, scatter_row, 0, unroll=8)
    pltpu.make_async_copy(mem_any.at[0], drain_scr, sem).wait()


def kernel(memory, veh_idx, veh_repr, cust_repr, edge_emb,
           W_in, b_in, W_h, b_h, ln_gamma, ln_beta):
    n, l, h = memory.shape
    d = veh_repr.shape[-1]
    x = jnp.concatenate(
        [veh_repr[:, 0, :], cust_repr[:, 0, :], edge_emb[:, 0, 0, :]], axis=-1)
    w_in_t = W_in.T
    w_h_t = W_h.T
    bias = (b_in + b_h).reshape(1, h)
    gamma = ln_gamma.reshape(1, h)
    beta = ln_beta.reshape(1, h)
    idx = veh_idx.reshape(n).astype(jnp.int32)

    grid_spec = pltpu.PrefetchScalarGridSpec(
        num_scalar_prefetch=1,
        grid=(),
        in_specs=[
            pl.BlockSpec(memory_space=pl.ANY),
            pl.BlockSpec(memory_space=pltpu.VMEM),
            pl.BlockSpec(memory_space=pltpu.VMEM),
            pl.BlockSpec(memory_space=pltpu.VMEM),
            pl.BlockSpec(memory_space=pltpu.VMEM),
            pl.BlockSpec(memory_space=pltpu.VMEM),
            pl.BlockSpec(memory_space=pltpu.VMEM),
        ],
        out_specs=pl.BlockSpec(memory_space=pltpu.VMEM),
        scratch_shapes=[
            pltpu.VMEM((n, h), jnp.float32),
            pltpu.SemaphoreType.DMA,
        ],
    )
    next_h = pl.pallas_call(
        _gather_mlp_body,
        grid_spec=grid_spec,
        out_shape=jax.ShapeDtypeStruct((n, h), jnp.float32),
    )(idx, memory, x, w_in_t, w_h_t, bias, gamma, beta)

    return pl.pallas_call(
        _scatter_body,
        in_specs=[
            pl.BlockSpec(memory_space=pl.ANY),
            pl.BlockSpec(memory_space=pltpu.VMEM),
            pl.BlockSpec(memory_space=pltpu.SMEM),
        ],
        out_specs=pl.BlockSpec(memory_space=pl.ANY),
        out_shape=jax.ShapeDtypeStruct((n, l, h), jnp.float32),
        scratch_shapes=[
            pltpu.VMEM((n, h), jnp.float32),
            pltpu.SemaphoreType.DMA,
        ],
        input_output_aliases={0: 0},
    )(memory, next_h, idx)


# speedup vs baseline: 1.2070x; 1.0024x over previous
import jax
import jax.numpy as jnp
from jax import lax
from jax.experimental import pallas as pl
from jax.experimental.pallas import tpu as pltpu


def _gather_mlp_body(idx_sref, mem_any, x_ref, w_in_t_ref, w_h_t_ref,
                     bias_ref, gamma_ref, beta_ref, nh_ref, curh_scr, sem):
    n = nh_ref.shape[0]

    def gather_row(r, _):
        idx = idx_sref[r]
        pltpu.make_async_copy(mem_any.at[r, idx], curh_scr.at[r], sem).start()
        return 0

    lax.fori_loop(0, n, gather_row, 0, unroll=8)
    pltpu.make_async_copy(mem_any.at[0], curh_scr, sem).wait()

    pre = (jnp.dot(x_ref[...], w_in_t_ref[...], preferred_element_type=jnp.float32)
           + jnp.dot(curh_scr[...], w_h_t_ref[...], preferred_element_type=jnp.float32)
           + bias_ref[...])
    mean = jnp.mean(pre, axis=-1, keepdims=True)
    cent = pre - mean
    var = jnp.mean(cent * cent, axis=-1, keepdims=True)
    nh_ref[...] = jnp.tanh(cent * lax.rsqrt(var + 1e-5) * gamma_ref[...]
                           + beta_ref[...])


def _scatter_body(mem_any, nh_ref, idx_sref, out_any, drain_scr, sem):
    n = nh_ref.shape[0]

    def scatter_row(r, _):
        idx = idx_sref[r]
        pltpu.make_async_copy(nh_ref.at[r], out_any.at[r, idx], sem).start()
        return 0

    lax.fori_loop(0, n, scatter_row, 0, unroll=8)
    pltpu.make_async_copy(mem_any.at[0], drain_scr, sem).wait()


def kernel(memory, veh_idx, veh_repr, cust_repr, edge_emb,
           W_in, b_in, W_h, b_h, ln_gamma, ln_beta):
    n, l, h = memory.shape
    d = veh_repr.shape[-1]
    x = jnp.concatenate(
        [veh_repr[:, 0, :], cust_repr[:, 0, :], edge_emb[:, 0, 0, :]], axis=-1)
    w_in_t = W_in.T
    w_h_t = W_h.T
    bias = (b_in + b_h).reshape(1, h)
    gamma = ln_gamma.reshape(1, h)
    beta = ln_beta.reshape(1, h)
    idx = veh_idx.reshape(n).astype(jnp.int32)

    grid_spec = pltpu.PrefetchScalarGridSpec(
        num_scalar_prefetch=1,
        grid=(),
        in_specs=[
            pl.BlockSpec(memory_space=pl.ANY),
            pl.BlockSpec(memory_space=pltpu.VMEM),
            pl.BlockSpec(memory_space=pltpu.VMEM),
            pl.BlockSpec(memory_space=pltpu.VMEM),
            pl.BlockSpec(memory_space=pltpu.VMEM),
            pl.BlockSpec(memory_space=pltpu.VMEM),
            pl.BlockSpec(memory_space=pltpu.VMEM),
        ],
        out_specs=pl.BlockSpec(memory_space=pltpu.VMEM),
        scratch_shapes=[
            pltpu.VMEM((n, h), jnp.float32),
            pltpu.SemaphoreType.DMA,
        ],
    )
    copy_ref = jax.new_ref(memory)
    next_h = pl.pallas_call(
        _gather_mlp_body,
        grid_spec=grid_spec,
        out_shape=jax.ShapeDtypeStruct((n, h), jnp.float32),
    )(idx, memory, x, w_in_t, w_h_t, bias, gamma, beta)
    base = jax.freeze(copy_ref)

    return pl.pallas_call(
        _scatter_body,
        in_specs=[
            pl.BlockSpec(memory_space=pl.ANY),
            pl.BlockSpec(memory_space=pltpu.VMEM),
            pl.BlockSpec(memory_space=pltpu.SMEM),
        ],
        out_specs=pl.BlockSpec(memory_space=pl.ANY),
        out_shape=jax.ShapeDtypeStruct((n, l, h), jnp.float32),
        scratch_shapes=[
            pltpu.VMEM((n, h), jnp.float32),
            pltpu.SemaphoreType.DMA,
        ],
        input_output_aliases={0: 0},
    )(base, next_h, idx)


# speedup vs baseline: 1.2297x; 1.0188x over previous
import jax
import jax.numpy as jnp
from jax import lax
from jax.experimental import pallas as pl
from jax.experimental.pallas import tpu as pltpu


def _fused_body(idx_sref, mem_any, x_ref, w_in_t_ref, w_h_t_ref,
                bias_ref, gamma_ref, beta_ref, out_any,
                curh_scr, nh_scr, sem_g, sem_s):
    n = nh_scr.shape[0]

    def gather_row(r, _):
        idx = idx_sref[r]
        pltpu.make_async_copy(mem_any.at[r, idx], curh_scr.at[r], sem_g).start()
        return 0

    lax.fori_loop(0, n, gather_row, 0, unroll=8)
    pltpu.make_async_copy(mem_any.at[0], curh_scr, sem_g).wait()

    pre = (jnp.dot(x_ref[...], w_in_t_ref[...], preferred_element_type=jnp.float32)
           + jnp.dot(curh_scr[...], w_h_t_ref[...], preferred_element_type=jnp.float32)
           + bias_ref[...])
    mean = jnp.mean(pre, axis=-1, keepdims=True)
    cent = pre - mean
    var = jnp.mean(cent * cent, axis=-1, keepdims=True)
    nh_scr[...] = jnp.tanh(cent * lax.rsqrt(var + 1e-5) * gamma_ref[...]
                           + beta_ref[...])

    def scatter_row(r, _):
        idx = idx_sref[r]
        pltpu.make_async_copy(nh_scr.at[r], out_any.at[r, idx], sem_s).start()
        return 0

    lax.fori_loop(0, n, scatter_row, 0, unroll=8)
    pltpu.make_async_copy(mem_any.at[0], curh_scr, sem_s).wait()


def kernel(memory, veh_idx, veh_repr, cust_repr, edge_emb,
           W_in, b_in, W_h, b_h, ln_gamma, ln_beta):
    n, l, h = memory.shape
    d = veh_repr.shape[-1]
    x = jnp.concatenate(
        [veh_repr[:, 0, :], cust_repr[:, 0, :], edge_emb[:, 0, 0, :]], axis=-1)
    w_in_t = W_in.T
    w_h_t = W_h.T
    bias = (b_in + b_h).reshape(1, h)
    gamma = ln_gamma.reshape(1, h)
    beta = ln_beta.reshape(1, h)
    idx = veh_idx.reshape(n).astype(jnp.int32)

    grid_spec = pltpu.PrefetchScalarGridSpec(
        num_scalar_prefetch=1,
        grid=(),
        in_specs=[
            pl.BlockSpec(memory_space=pl.ANY),
            pl.BlockSpec(memory_space=pltpu.VMEM),
            pl.BlockSpec(memory_space=pltpu.VMEM),
            pl.BlockSpec(memory_space=pltpu.VMEM),
            pl.BlockSpec(memory_space=pltpu.VMEM),
            pl.BlockSpec(memory_space=pltpu.VMEM),
            pl.BlockSpec(memory_space=pltpu.VMEM),
        ],
        out_specs=pl.BlockSpec(memory_space=pl.ANY),
        scratch_shapes=[
            pltpu.VMEM((n, h), jnp.float32),
            pltpu.VMEM((n, h), jnp.float32),
            pltpu.SemaphoreType.DMA,
            pltpu.SemaphoreType.DMA,
        ],
    )
    return pl.pallas_call(
        _fused_body,
        grid_spec=grid_spec,
        out_shape=jax.ShapeDtypeStruct((n, l, h), jnp.float32),
        input_output_aliases={1: 0},
    )(idx, memory, x, w_in_t, w_h_t, bias, gamma, beta)
